# baseline (speedup 1.0000x reference)
import functools

import jax
import jax.numpy as jnp
from jax.experimental import pallas as pl
from jax.experimental.pallas import tpu as pltpu

B = 8
IN = 512
H = 1024
N = 32768
W = 64
K = 4

NB = 2048
GRID = N // NB

_INTERPRET = False


def _ctrl_body(x_ref, r_ref, h_ref, c_ref, wx_ref, wr_ref, wh_ref, b_ref,
               pw_ref, pb_ref, h_out, c_out, qn_out, a_out, e_out, ag_out):
    z = (jnp.dot(x_ref[...], wx_ref[...], preferred_element_type=jnp.float32)
         + jnp.dot(r_ref[...], wr_ref[...], preferred_element_type=jnp.float32)
         + jnp.dot(h_ref[...], wh_ref[...], preferred_element_type=jnp.float32)
         + b_ref[...])
    zi = z[:, :H]
    zf = z[:, H:2 * H]
    zg = z[:, 2 * H:3 * H]
    zo = z[:, 3 * H:]
    i_g = jax.nn.sigmoid(zi)
    f_g = jax.nn.sigmoid(zf)
    g_g = jnp.tanh(zg)
    o_g = jax.nn.sigmoid(zo)
    c = f_g * c_ref[...] + i_g * g_g
    h = o_g * jnp.tanh(c)
    h_out[...] = h
    c_out[...] = c
    params = jnp.dot(h, pw_ref[...], preferred_element_type=jnp.float32) + pb_ref[...]
    q = params[:, :W]
    a = params[:, W:2 * W]
    e = jax.nn.sigmoid(params[:, 2 * W:3 * W])
    alpha = jax.nn.sigmoid(params[:, 3 * W:3 * W + 1])
    gamma = jax.nn.sigmoid(params[:, 3 * W + 1:3 * W + 2])
    qn_out[...] = q * jax.lax.rsqrt(
        jnp.maximum(jnp.sum(q * q, axis=1, keepdims=True), 1e-12))
    a_out[...] = a
    e_out[...] = e
    ci = jax.lax.broadcasted_iota(jnp.int32, (B, 128), 1)
    ag = jnp.where(ci == 0, alpha,
                   jnp.where(ci == 1, gamma,
                             jnp.where(ci == 2, alpha * gamma,
                                       jnp.where(ci == 3, alpha * (1.0 - gamma),
                                                 0.0))))
    ag_out[...] = ag


def _stream_body(m_ref, wr_ref, qn_ref, a_ref, e_ref, ag_ref, sim_out, m_out):
    m = m_ref[...]
    inv = jax.lax.rsqrt(
        jnp.maximum(jnp.sum(m * m, axis=2, keepdims=True), 1e-12))
    sim_out[...] = jnp.sum(m * qn_ref[...][:, None, :], axis=2) * inv[:, :, 0]
    w1 = ag_ref[:, 2:3] * wr_ref[...]
    w1e = w1[:, :, None] * e_ref[...][:, None, :]
    m_out[...] = m * (1.0 - w1e) + w1[:, :, None] * a_ref[...][:, None, :]


def _route_body(sim_ref, use_ref, wrp_ref, ag_ref,
                wr_out, use_out, mi_out, mf_out):
    s = sim_ref[...]
    col = jax.lax.broadcasted_iota(jnp.int32, (B, N), 1)
    idxs = []
    vals = []
    for _ in range(K):
        v = jnp.max(s, axis=1, keepdims=True)
        idx = jnp.min(jnp.where(s == v, col, N), axis=1, keepdims=True)
        idxs.append(idx)
        vals.append(v)
        s = jnp.where(col == idx, -jnp.inf, s)
    exps = [jnp.exp(v - vals[0]) for v in vals]
    denom = exps[0]
    for t in exps[1:]:
        denom = denom + t
    w_r = jnp.zeros((B, N), dtype=jnp.float32)
    access = jnp.zeros((B, N), dtype=jnp.float32)
    for idx, ex in zip(idxs, exps):
        hit = col == idx
        w_r = jnp.where(hit, ex / denom, w_r)
        access = jnp.where(hit, 1.0, access)
    wr_out[...] = w_r
    usage = (use_ref[...] + 1.0) * (1.0 - access)
    use_out[...] = usage
    um = jnp.max(usage, axis=1, keepdims=True)
    lru = jnp.min(jnp.where(usage == um, col, N), axis=1, keepdims=True)
    wr_at_lru = jnp.sum(jnp.where(col == lru, wrp_ref[...], 0.0),
                        axis=1, keepdims=True)
    w2 = ag_ref[:, 2:3] * wr_at_lru + ag_ref[:, 3:4]
    ci = jax.lax.broadcasted_iota(jnp.int32, (B, 128), 1)
    mi = jnp.zeros((B, 128), jnp.int32)
    mf = jnp.zeros((B, 128), jnp.float32)
    for k, idx in enumerate(idxs):
        mi = jnp.where(ci == k, jnp.broadcast_to(idx, (B, 128)), mi)
    mi = jnp.where(ci == K, jnp.broadcast_to(lru, (B, 128)), mi)
    for k, ex in enumerate(exps):
        mf = jnp.where(ci == k, jnp.broadcast_to(ex / denom, (B, 128)), mf)
    mf = jnp.where(ci == K, jnp.broadcast_to(w2, (B, 128)), mf)
    mi_out[...] = mi
    mf_out[...] = mf


def _rows_body(idx_ref, r0, r1, r2, r3, r4, e_ref, a_ref, w_ref, mprov,
               r_out, mfix_out):
    b = pl.program_id(0)
    del mprov
    acc = (w_ref[b, 0] * r0[...] + w_ref[b, 1] * r1[...]
           + w_ref[b, 2] * r2[...] + w_ref[b, 3] * r3[...])
    r_out[...] = acc.reshape(1, 1, W)
    w2 = w_ref[b, K]
    row = r4[...].reshape(1, 1, W)
    fixed = row * (1.0 - w2 * e_ref[...]) + w2 * a_ref[...]
    mfix_out[...] = fixed.reshape(1, 1, 1, W)


def _final_body(h_ref, r_ref, w1_ref, w2_ref, b_ref, y_out):
    y_out[...] = (jnp.dot(h_ref[...], w1_ref[...], preferred_element_type=jnp.float32)
                  + jnp.dot(r_ref[...], w2_ref[...], preferred_element_type=jnp.float32)
                  + b_ref[...])


@jax.jit
def kernel(inputs, h_prev, c_prev, M_prev, wr_prev, usage_prev, r_prev,
           lstm_kernel, lstm_rec_kernel, lstm_bias, proj_W, proj_b,
           final_W, final_b):
    f32 = jnp.float32

    h_curr, c_curr, qn, a_, e_, ag = pl.pallas_call(
        _ctrl_body,
        out_shape=[
            jax.ShapeDtypeStruct((B, H), f32),
            jax.ShapeDtypeStruct((B, H), f32),
            jax.ShapeDtypeStruct((B, W), f32),
            jax.ShapeDtypeStruct((B, W), f32),
            jax.ShapeDtypeStruct((B, W), f32),
            jax.ShapeDtypeStruct((B, 128), f32),
        ],
        interpret=_INTERPRET,
    )(inputs, r_prev, h_prev, c_prev,
      lstm_kernel[:IN], lstm_kernel[IN:], lstm_rec_kernel,
      lstm_bias.reshape(1, 4 * H), proj_W, proj_b.reshape(1, 3 * W + 2))

    sim, m_prov = pl.pallas_call(
        _stream_body,
        grid=(GRID,),
        in_specs=[
            pl.BlockSpec((B, NB, W), lambda i: (0, i, 0)),
            pl.BlockSpec((B, NB), lambda i: (0, i)),
            pl.BlockSpec((B, W), lambda i: (0, 0)),
            pl.BlockSpec((B, W), lambda i: (0, 0)),
            pl.BlockSpec((B, W), lambda i: (0, 0)),
            pl.BlockSpec((B, 128), lambda i: (0, 0)),
        ],
        out_specs=[
            pl.BlockSpec((B, NB), lambda i: (0, i)),
            pl.BlockSpec((B, NB, W), lambda i: (0, i, 0)),
        ],
        out_shape=[
            jax.ShapeDtypeStruct((B, N), f32),
            jax.ShapeDtypeStruct((B, N, W), f32),
        ],
        interpret=_INTERPRET,
    )(M_prev, wr_prev, qn, a_, e_, ag)

    w_r, usage_curr, mi, mf = pl.pallas_call(
        _route_body,
        out_shape=[
            jax.ShapeDtypeStruct((B, N), f32),
            jax.ShapeDtypeStruct((B, N), f32),
            jax.ShapeDtypeStruct((B, 128), jnp.int32),
            jax.ShapeDtypeStruct((B, 128), f32),
        ],
        interpret=_INTERPRET,
    )(sim, usage_prev, wr_prev, ag)

    idx5 = mi[:, :K + 1].reshape(B * (K + 1))
    w5 = mf[:, :8]

    M_prev4 = M_prev.reshape(B, N, 1, W)
    row_spec = [
        pl.BlockSpec((1, 1, 1, W), functools.partial(
            lambda k, b, iref: (b, iref[b * (K + 1) + k], 0, 0), k))
        for k in range(K + 1)
    ]
    r3, M_curr4 = pl.pallas_call(
        _rows_body,
        grid_spec=pltpu.PrefetchScalarGridSpec(
            num_scalar_prefetch=1,
            grid=(B,),
            in_specs=row_spec + [
                pl.BlockSpec((1, 1, W), lambda b, iref: (b, 0, 0)),
                pl.BlockSpec((1, 1, W), lambda b, iref: (b, 0, 0)),
                pl.BlockSpec(memory_space=pltpu.SMEM),
                pl.BlockSpec(memory_space=pl.ANY),
            ],
            out_specs=[
                pl.BlockSpec((1, 1, W), lambda b, iref: (b, 0, 0)),
                pl.BlockSpec((1, 1, 1, W),
                             lambda b, iref: (b, iref[b * (K + 1) + K], 0, 0)),
            ],
        ),
        out_shape=[
            jax.ShapeDtypeStruct((B, 1, W), f32),
            jax.ShapeDtypeStruct((B, N, 1, W), f32),
        ],
        input_output_aliases={9: 1},
        interpret=_INTERPRET,
    )(idx5, M_prev4, M_prev4, M_prev4, M_prev4, M_prev4,
      e_.reshape(B, 1, W), a_.reshape(B, 1, W), w5,
      m_prov.reshape(B, N, 1, W))

    M_curr = M_curr4.reshape(B, N, W)
    r_curr = r3.reshape(B, W)

    y_out = pl.pallas_call(
        _final_body,
        out_shape=jax.ShapeDtypeStruct((B, H), f32),
        interpret=_INTERPRET,
    )(h_curr, r_curr, final_W[:H], final_W[H:], final_b.reshape(1, H))

    return (y_out, (h_curr, c_curr, M_curr, w_r, usage_curr, r_curr))
